# Initial kernel scaffold; baseline (speedup 1.0000x reference)
import jax
import jax.numpy as jnp
from jax.experimental import pallas as pl


def kernel(x, var):
    raise NotImplementedError("write your pallas kernel here")


# speedup vs baseline: 3.3349x; 3.3349x over previous
import functools

import jax
import jax.numpy as jnp
from jax import lax
from jax.experimental import pallas as pl
from jax.experimental.pallas import tpu as pltpu
from jax.experimental.pallas import tpu_sc as plsc

VOCAB = 100000
DIM = 128
BATCH = 4096
SEQ = 50
N = BATCH * SEQ
NC = 2
NS = 16
NW = NC * NS
PER_W = N // NW
CHUNK = 128
NCHUNK = PER_W // CHUNK

_mesh = plsc.VectorSubcoreMesh(
    core_axis_name="c", subcore_axis_name="s", num_cores=NC, num_subcores=NS
)


@functools.partial(
    pl.kernel,
    out_type=jax.ShapeDtypeStruct((N, DIM), jnp.float32),
    mesh=_mesh,
    scratch_types=[
        pltpu.VMEM((PER_W,), jnp.int32),
        pltpu.VMEM((CHUNK, DIM), jnp.float32),
        pltpu.VMEM((CHUNK, DIM), jnp.float32),
        pltpu.SemaphoreType.DMA,
        pltpu.SemaphoreType.DMA,
    ],
)
def _emb_lookup(x_hbm, var_hbm, out_hbm, idx_v, buf0, buf1, sem0, sem1):
    wid = lax.axis_index("s") * NC + lax.axis_index("c")
    base = wid * PER_W
    pltpu.sync_copy(x_hbm.at[pl.ds(base, PER_W)], idx_v)

    bufs = (buf0, buf1)
    sems = (sem0, sem1)

    def gather(c, b):
        return pltpu.async_copy(
            var_hbm.at[idx_v.at[pl.ds(c * CHUNK, CHUNK)]], bufs[b], sems[b]
        )

    gather(0, 0)
    gather(1, 1)

    @pl.loop(0, NCHUNK, step=2)
    def _(c):
        for b in range(2):
            cc = c + b
            pltpu.make_async_copy(
                var_hbm.at[idx_v.at[pl.ds(cc * CHUNK, CHUNK)]], bufs[b], sems[b]
            ).wait()
            pltpu.sync_copy(bufs[b], out_hbm.at[pl.ds(base + cc * CHUNK, CHUNK)])
            nxt = cc + 2

            @pl.when(nxt < NCHUNK)
            def _():
                gather(nxt, b)


def kernel(x, var):
    flat = _emb_lookup(x.reshape(N).astype(jnp.int32), var)
    return flat.reshape(BATCH, SEQ, DIM)
